# baseline (speedup 1.0000x reference)
import functools

import jax
import jax.numpy as jnp
from jax.experimental import pallas as pl
from jax.experimental.pallas import tpu as pltpu


def _fused_rgcn_kernel(cols_ref, log_ref, adj_ref, pw_ref, b_ref,
                       g_ref, bt_ref, w_ref, v_ref, out_ref,
                       acc0_s, yb_s,
                       *, H, L, R, K3, N, BE, TN):
    t = pl.program_id(1)

    @pl.when(t == 0)
    def _pass1():
        wb = w_ref[...].astype(jnp.bfloat16)
        m0 = wb[:, 0:H] * v_ref[0:1, 0:1].astype(jnp.bfloat16)
        for k in range(1, K3):
            m0 = m0 + wb[:, k * H:(k + 1) * H] * v_ref[0:1, k:k + 1].astype(
                jnp.bfloat16)
        wbig = jnp.concatenate([m0, wb], axis=1)
        cdims = (((1,), (1,)), ((), ()))
        for e in range(BE):
            cols = cols_ref[e]
            z = jax.lax.dot_general(cols, pw_ref[:, :H], cdims,
                                    preferred_element_type=jnp.float32)
            z = z + jax.lax.dot_general(log_ref[e], pw_ref[:, H:], cdims,
                                        preferred_element_type=jnp.float32)
            z = z + b_ref[...]
            inv_h = 1.0 / H
            mu = jnp.sum(z, axis=-1, keepdims=True) * inv_h
            dm = z - mu
            var = jnp.sum(dm * dm, axis=-1, keepdims=True) * inv_h
            xh = dm * jax.lax.rsqrt(var + 1e-5)
            xh = xh * g_ref[...] + bt_ref[...]
            x = 0.5 * xh * (1.0 + jax.lax.erf(xh * 0.7071067811865475))
            xb = x.astype(jnp.bfloat16)
            yfull = jnp.dot(xb, wbig, preferred_element_type=jnp.float32)
            acc0_s[e] = yfull[:, 0:H]
            yb_s[e] = yfull[:, H:].astype(jnp.bfloat16)

    tbls = []
    for k in range(K3):
        tbl = jnp.where(jax.lax.broadcasted_iota(jnp.int32, (R, 1), 0) == 0,
                        0.0, v_ref[:, k:k + 1])
        tbls.append(jnp.broadcast_to(tbl, (R, N)))
    row0 = pl.multiple_of(t * TN, TN)
    for e in range(BE):
        adj = adj_ref[e]
        acc = acc0_s[e, pl.ds(row0, TN), :]
        yb = yb_s[e]
        for k in range(K3):
            d = jnp.take_along_axis(tbls[k], adj, axis=0).astype(jnp.bfloat16)
            acc = acc + jnp.dot(d, yb[:, k * H:(k + 1) * H],
                                preferred_element_type=jnp.float32)
        out_ref[e] = acc.astype(out_ref.dtype)


def kernel(columns, logits, adj, proj_w, proj_b, ln_g, ln_b, W, V):
    B, N, H = columns.shape
    L = logits.shape[-1]
    R, K3 = V.shape

    bias = proj_b.reshape(1, H)
    gamma = ln_g.reshape(1, H)
    beta = ln_b.reshape(1, H)
    w2d = W.reshape(H, K3 * H)

    BE = 4 if B % 4 == 0 else (2 if B % 2 == 0 else 1)
    TN = 128 if N % 128 == 0 else N
    NT = N // TN

    flops = 2 * B * N * (H * H + K3 * H * H + K3 * N * H)
    cost = pl.CostEstimate(
        flops=int(flops),
        transcendentals=int(B * N * H),
        bytes_accessed=int(B * N * N * 4 + 2 * B * N * H * 4 + B * N * L * 4),
    )

    out = pl.pallas_call(
        functools.partial(_fused_rgcn_kernel, H=H, L=L, R=R, K3=K3, N=N,
                          BE=BE, TN=TN),
        out_shape=jax.ShapeDtypeStruct((B, N, H), columns.dtype),
        grid=(B // BE, NT),
        in_specs=[
            pl.BlockSpec((BE, N, H), lambda g, t: (g, 0, 0)),
            pl.BlockSpec((BE, N, L), lambda g, t: (g, 0, 0)),
            pl.BlockSpec((BE, TN, N), lambda g, t: (g, t, 0)),
            pl.BlockSpec((H, H + L), lambda g, t: (0, 0)),
            pl.BlockSpec((1, H), lambda g, t: (0, 0)),
            pl.BlockSpec((1, H), lambda g, t: (0, 0)),
            pl.BlockSpec((1, H), lambda g, t: (0, 0)),
            pl.BlockSpec((H, K3 * H), lambda g, t: (0, 0)),
            pl.BlockSpec((R, K3), lambda g, t: (0, 0)),
        ],
        out_specs=pl.BlockSpec((BE, TN, H), lambda g, t: (g, t, 0)),
        scratch_shapes=[
            pltpu.VMEM((BE, N, H), jnp.float32),
            pltpu.VMEM((BE, N, K3 * H), jnp.bfloat16),
        ],
        compiler_params=pltpu.CompilerParams(
            dimension_semantics=("parallel", "arbitrary")),
        cost_estimate=cost,
    )(columns, logits, adj, proj_w, bias, gamma, beta, w2d, V)
    return out


# speedup vs baseline: 1.2326x; 1.2326x over previous
import functools

import jax
import jax.numpy as jnp
from jax.experimental import pallas as pl
from jax.experimental.pallas import tpu as pltpu


def _fused_rgcn_kernel(cols_ref, log_ref, adj_ref, pw_ref, b_ref,
                       g_ref, bt_ref, w_ref, v_ref, out_ref,
                       *, H, L, R, K3, N, BE):
    wb = w_ref[...].astype(jnp.bfloat16)
    m0 = wb[:, 0:H] * v_ref[0:1, 0:1].astype(jnp.bfloat16)
    for k in range(1, K3):
        m0 = m0 + wb[:, k * H:(k + 1) * H] * v_ref[0:1, k:k + 1].astype(jnp.bfloat16)
    wbig = jnp.concatenate([m0, wb], axis=1)
    cdims = (((1,), (1,)), ((), ()))

    for e in range(BE):
        cols = cols_ref[e]
        z = jax.lax.dot_general(cols, pw_ref[:, :H], cdims,
                                preferred_element_type=jnp.float32)
        z = z + jax.lax.dot_general(log_ref[e], pw_ref[:, H:], cdims,
                                    preferred_element_type=jnp.float32)
        z = z + b_ref[...]
        inv_h = 1.0 / H
        mu = jnp.sum(z, axis=-1, keepdims=True) * inv_h
        dm = z - mu
        var = jnp.sum(dm * dm, axis=-1, keepdims=True) * inv_h
        xh = dm * jax.lax.rsqrt(var + 1e-5)
        xh = xh * g_ref[...] + bt_ref[...]
        x = 0.5 * xh * (1.0 + jax.lax.erf(xh * 0.7071067811865475))
        xb = x.astype(jnp.bfloat16)

        yfull = jnp.dot(xb, wbig, preferred_element_type=jnp.float32)
        acc = yfull[:, 0:H]
        yb = yfull[:, H:].astype(jnp.bfloat16)

        adj = adj_ref[e]
        d = []
        for k in range(K3):
            tbl = jnp.where(jax.lax.broadcasted_iota(jnp.int32, (R, 1), 0) == 0,
                            0.0, v_ref[:, k:k + 1])
            tbl_bc = jnp.broadcast_to(tbl, (R, N))
            d.append(jnp.take_along_axis(tbl_bc, adj, axis=0)
                     .astype(jnp.bfloat16))

        for k in range(K3):
            acc = acc + jnp.dot(d[k], yb[:, k * H:(k + 1) * H],
                                preferred_element_type=jnp.float32)
        out_ref[e] = acc.astype(out_ref.dtype)


def kernel(columns, logits, adj, proj_w, proj_b, ln_g, ln_b, W, V):
    B, N, H = columns.shape
    L = logits.shape[-1]
    R, K3 = V.shape

    bias = proj_b.reshape(1, H)
    gamma = ln_g.reshape(1, H)
    beta = ln_b.reshape(1, H)
    w2d = W.reshape(H, K3 * H)

    flops = 2 * B * N * (H * H + K3 * H * H + K3 * N * H)
    cost = pl.CostEstimate(
        flops=int(flops),
        transcendentals=int(B * N * H),
        bytes_accessed=int(B * N * N * 4 + 2 * B * N * H * 4 + B * N * L * 4),
    )

    BE = 8 if B % 8 == 0 else (2 if B % 2 == 0 else 1)
    out = pl.pallas_call(
        functools.partial(_fused_rgcn_kernel, H=H, L=L, R=R, K3=K3, N=N, BE=BE),
        out_shape=jax.ShapeDtypeStruct((B, N, H), columns.dtype),
        grid=(B // BE,),
        in_specs=[
            pl.BlockSpec((BE, N, H), lambda b: (b, 0, 0)),
            pl.BlockSpec((BE, N, L), lambda b: (b, 0, 0)),
            pl.BlockSpec((BE, N, N), lambda b: (b, 0, 0)),
            pl.BlockSpec((H, H + L), lambda b: (0, 0)),
            pl.BlockSpec((1, H), lambda b: (0, 0)),
            pl.BlockSpec((1, H), lambda b: (0, 0)),
            pl.BlockSpec((1, H), lambda b: (0, 0)),
            pl.BlockSpec((H, K3 * H), lambda b: (0, 0)),
            pl.BlockSpec((R, K3), lambda b: (0, 0)),
        ],
        out_specs=pl.BlockSpec((BE, N, H), lambda b: (b, 0, 0)),
        compiler_params=pltpu.CompilerParams(
            dimension_semantics=("parallel",)),
        cost_estimate=cost,
    )(columns, logits, adj, proj_w, bias, gamma, beta, w2d, V)
    return out


# speedup vs baseline: 1.2890x; 1.0458x over previous
import functools

import jax
import jax.numpy as jnp
from jax.experimental import pallas as pl
from jax.experimental.pallas import tpu as pltpu


def _fused_rgcn_kernel(cols_ref, log_ref, adj_ref, pw_ref, b_ref,
                       g_ref, bt_ref, w_ref, v_ref, out_ref,
                       *, H, L, R, K3, N, BE):
    wb = w_ref[...].astype(jnp.bfloat16)
    m0 = wb[:, 0:H] * v_ref[0:1, 0:1].astype(jnp.bfloat16)
    for k in range(1, K3):
        m0 = m0 + wb[:, k * H:(k + 1) * H] * v_ref[0:1, k:k + 1].astype(jnp.bfloat16)
    wbig = jnp.concatenate([m0, wb], axis=1)
    cdims = (((1,), (1,)), ((), ()))

    for e in range(BE):
        cols = cols_ref[e]
        z = jax.lax.dot_general(cols, pw_ref[:, :H], cdims,
                                preferred_element_type=jnp.float32)
        z = z + jax.lax.dot_general(log_ref[e], pw_ref[:, H:], cdims,
                                    preferred_element_type=jnp.float32)
        z = z + b_ref[...]
        inv_h = 1.0 / H
        mu = jnp.sum(z, axis=-1, keepdims=True) * inv_h
        dm = z - mu
        var = jnp.sum(dm * dm, axis=-1, keepdims=True) * inv_h
        xh = dm * jax.lax.rsqrt(var + 1e-5)
        xh = xh * g_ref[...] + bt_ref[...]
        x = 0.5 * xh * (1.0 + jax.lax.erf(xh * 0.7071067811865475))
        xb = x.astype(jnp.bfloat16)

        yfull = jnp.dot(xb, wbig, preferred_element_type=jnp.float32)
        acc = yfull[:, 0:H]
        yb = yfull[:, H:].astype(jnp.bfloat16)

        adj = adj_ref[e]
        d = []
        for k in range(K3):
            tbl = jnp.where(jax.lax.broadcasted_iota(jnp.int32, (R, 1), 0) == 0,
                            0.0, v_ref[:, k:k + 1])
            tbl_bc = jnp.broadcast_to(tbl, (R, N))
            d.append(jnp.take_along_axis(tbl_bc, adj, axis=0)
                     .astype(jnp.bfloat16))

        for k in range(K3):
            acc = acc + jnp.dot(d[k], yb[:, k * H:(k + 1) * H],
                                preferred_element_type=jnp.float32)
        out_ref[e] = acc.astype(out_ref.dtype)


def kernel(columns, logits, adj, proj_w, proj_b, ln_g, ln_b, W, V):
    B, N, H = columns.shape
    L = logits.shape[-1]
    R, K3 = V.shape

    bias = proj_b.reshape(1, H)
    gamma = ln_g.reshape(1, H)
    beta = ln_b.reshape(1, H)
    w2d = W.reshape(H, K3 * H)

    flops = 2 * B * N * (H * H + K3 * H * H + K3 * N * H)
    cost = pl.CostEstimate(
        flops=int(flops),
        transcendentals=int(B * N * H),
        bytes_accessed=int(B * N * N * 4 + 2 * B * N * H * 4 + B * N * L * 4),
    )

    BE = 4 if B % 4 == 0 else (2 if B % 2 == 0 else 1)
    out = pl.pallas_call(
        functools.partial(_fused_rgcn_kernel, H=H, L=L, R=R, K3=K3, N=N, BE=BE),
        out_shape=jax.ShapeDtypeStruct((B, N, H), columns.dtype),
        grid=(B // BE,),
        in_specs=[
            pl.BlockSpec((BE, N, H), lambda b: (b, 0, 0)),
            pl.BlockSpec((BE, N, L), lambda b: (b, 0, 0)),
            pl.BlockSpec((BE, N, N), lambda b: (b, 0, 0)),
            pl.BlockSpec((H, H + L), lambda b: (0, 0)),
            pl.BlockSpec((1, H), lambda b: (0, 0)),
            pl.BlockSpec((1, H), lambda b: (0, 0)),
            pl.BlockSpec((1, H), lambda b: (0, 0)),
            pl.BlockSpec((H, K3 * H), lambda b: (0, 0)),
            pl.BlockSpec((R, K3), lambda b: (0, 0)),
        ],
        out_specs=pl.BlockSpec((BE, N, H), lambda b: (b, 0, 0)),
        compiler_params=pltpu.CompilerParams(
            dimension_semantics=("parallel",)),
        cost_estimate=cost,
    )(columns, logits, adj, proj_w, bias, gamma, beta, w2d, V)
    return out
